# baseline (speedup 1.0000x reference)
import functools

import jax
import jax.numpy as jnp
from jax import lax
from jax.experimental import pallas as pl
from jax.experimental.pallas import tpu as pltpu
from jax.experimental.pallas import tpu_sc as plsc

B, S, D = 4, 8192, 2048
L = 16


def _pooler(h_rows, tokens):
    mesh = plsc.VectorSubcoreMesh(core_axis_name="c", subcore_axis_name="s",
                                  num_cores=1)

    @functools.partial(
        pl.kernel,
        out_type=jax.ShapeDtypeStruct((B, D), jnp.float32),
        mesh=mesh,
        compiler_params=pltpu.CompilerParams(needs_layout_passes=False,
                                             skip_device_barrier=True),
        scratch_types=[
            pltpu.VMEM((S,), jnp.int32),
            pltpu.SemaphoreType.DMA((4,)),
        ],
    )
    def k(h_hbm, tok_hbm, out_hbm, row_v, sems):
        sid = lax.axis_index("s")

        @pl.when(sid < B)
        def _():
            b = sid
            P = S // 4
            copies = [
                pltpu.async_copy(tok_hbm.at[b, pl.ds(p * P, P)],
                                 row_v.at[pl.ds(p * P, P)], sems.at[p])
                for p in range(4)
            ]

            U = 8

            def piece_body(base0):
                def body(i, acc):
                    base = base0 + i * (L * U)
                    for u in range(U):
                        x = row_v[pl.ds(base + u * L, L)]
                        acc = acc + (x != 0).astype(jnp.int32)
                    return acc
                return body

            lane_cnt = jnp.zeros((L,), jnp.int32)
            for p in range(4):
                copies[p].wait()
                lane_cnt = lax.fori_loop(0, P // (L * U),
                                         piece_body(p * P), lane_cnt)
            cnt = jnp.sum(lane_cnt)
            idx = cnt - 1
            idx = jnp.where(idx < 0, idx + S, idx)
            pltpu.sync_copy(h_hbm.at[b * S + idx], out_hbm.at[b])

    return k(h_rows, tokens)


def kernel(h, inputs):
    return _pooler(h.reshape(B * S, D), inputs)
